# Initial kernel scaffold; baseline (speedup 1.0000x reference)
import functools

import jax
import jax.numpy as jnp
from jax import lax
from jax.experimental import pallas as pl
from jax.experimental.pallas import tpu as pltpu
from jax.experimental.pallas import tpu_sc as plsc

D = 128
D_EDGE = 16
NC, NS, L = 2, 16, 16
NW = NC * NS
CH = 128


def _ew_body(ef_ref, we_ref, be_ref, out_ref):
    out_ref[...] = jnp.sum(ef_ref[...] * we_ref[...], axis=1) + be_ref[0]


def _compute_ew(edge_feats, W_e, b_e):
    E = edge_feats.shape[0]
    BE = 2000
    assert E % BE == 0
    return pl.pallas_call(
        _ew_body,
        grid=(E // BE,),
        in_specs=[pl.BlockSpec((BE, D_EDGE), lambda i: (i, 0)),
                  pl.BlockSpec((1, D_EDGE), lambda i: (0, 0)),
                  pl.BlockSpec((1,), lambda i: (0,))],
        out_specs=pl.BlockSpec((BE,), lambda i: (i,)),
        out_shape=jax.ShapeDtypeStruct((E,), jnp.float32),
    )(edge_feats, W_e, b_e)


def _sc_degrees(src3, dst3, ew3, npad):
    cpt = src3.shape[1]
    rows_pt = npad // NS
    mesh = plsc.VectorSubcoreMesh(core_axis_name="c", subcore_axis_name="s")

    @functools.partial(
        pl.kernel,
        out_type=jax.ShapeDtypeStruct((NC, 2, npad, 16), jnp.float32),
        mesh=mesh,
        scratch_types=[
            pltpu.VMEM((cpt, CH), jnp.int32),
            pltpu.VMEM((cpt, CH), jnp.int32),
            pltpu.VMEM((cpt, CH), jnp.float32),
            pltpu.VMEM((CH, 16), jnp.float32),
            pltpu.VMEM((npad // NS, 16), jnp.float32),
            pltpu.VMEM_SHARED((npad, 16), jnp.float32),
            pltpu.VMEM_SHARED((npad, 16), jnp.float32),
        ],
    )
    def deg_kernel(src_h, dst_h, ew_h, tabs_h,
                   src_v, dst_v, ew_v, valbuf, zbuf, tsrc, tdst):
        cid = lax.axis_index("c")
        sid = lax.axis_index("s")
        wid = cid * NS + sid
        pltpu.sync_copy(src_h.at[wid], src_v)
        pltpu.sync_copy(dst_h.at[wid], dst_v)
        pltpu.sync_copy(ew_h.at[wid], ew_v)

        zv = jnp.zeros((L,), jnp.float32)

        def zrow(i, _):
            zbuf[i] = zv
            return 0
        lax.fori_loop(0, rows_pt, zrow, 0)
        base = sid * rows_pt
        pltpu.sync_copy(zbuf, tsrc.at[pl.ds(base, rows_pt)])
        pltpu.sync_copy(zbuf, tdst.at[pl.ds(base, rows_pt)])

        onehot1 = jnp.where(lax.iota(jnp.int32, L) == 1, 1.0, 0.0)

        def vrow(i, _):
            valbuf[i] = onehot1
            return 0
        lax.fori_loop(0, CH, vrow, 0)

        plsc.subcore_barrier()

        col0 = jnp.zeros((L,), jnp.int32)

        def chunk(j, _):
            for r in range(CH // L):
                v = ew_v[j, pl.ds(r * L, L)]
                rows = lax.iota(jnp.int32, L) + r * L
                plsc.store_scatter(valbuf, [rows, col0], v)
            pltpu.sync_copy(valbuf, tsrc.at[src_v.at[j]], add=True)
            pltpu.sync_copy(valbuf, tdst.at[dst_v.at[j]], add=True)
            return 0
        lax.fori_loop(0, cpt, chunk, 0)

        plsc.subcore_barrier()

        pltpu.sync_copy(tsrc.at[pl.ds(base, rows_pt)], zbuf)
        pltpu.sync_copy(zbuf, tabs_h.at[cid, 0, pl.ds(base, rows_pt)])
        pltpu.sync_copy(tdst.at[pl.ds(base, rows_pt)], zbuf)
        pltpu.sync_copy(zbuf, tabs_h.at[cid, 1, pl.ds(base, rows_pt)])

    return deg_kernel(src3, dst3, ew3)


def _norms_body(tabs_ref, x_ref, ns_ref, nd_ref, nin_ref, nout_ref, x1_ref):
    t = tabs_ref[0] + tabs_ref[1]
    out_w = t[0, :, 0]
    deg_o = t[0, :, 1]
    in_w = t[1, :, 0]
    deg_i = t[1, :, 1]
    ns_ref[...] = lax.rsqrt(out_w)
    nd_ref[...] = lax.rsqrt(in_w)
    nin_ref[...] = lax.rsqrt(jnp.maximum(deg_i, 1.0))
    nout = lax.rsqrt(jnp.maximum(deg_o, 1.0))
    nout_ref[...] = nout
    x1_ref[...] = x_ref[...] * nout[:, None]


def _compute_norms(tabs, xp):
    npad = xp.shape[0]
    BN = 1024
    vec = jax.ShapeDtypeStruct((npad,), jnp.float32)
    return pl.pallas_call(
        _norms_body,
        grid=(npad // BN,),
        in_specs=[pl.BlockSpec((NC, 2, BN, 16), lambda i: (0, 0, i, 0)),
                  pl.BlockSpec((BN, D), lambda i: (i, 0))],
        out_specs=[pl.BlockSpec((BN,), lambda i: (i,))] * 4
        + [pl.BlockSpec((BN, D), lambda i: (i, 0))],
        out_shape=[vec, vec, vec, vec,
                   jax.ShapeDtypeStruct((npad, D), jnp.float32)],
    )(tabs, xp)


def _sc_spmm(x_h, src3, dst3, ew3, ns, nd, npad):
    cpt = src3.shape[1]
    rows_pt = npad // NS
    ZB = 64
    nz = rows_pt // ZB
    mesh = plsc.VectorSubcoreMesh(core_axis_name="c", subcore_axis_name="s")

    @functools.partial(
        pl.kernel,
        out_type=jax.ShapeDtypeStruct((NC, npad, D), jnp.float32),
        mesh=mesh,
        scratch_types=[
            pltpu.VMEM((cpt, CH), jnp.int32),
            pltpu.VMEM((cpt, CH), jnp.int32),
            pltpu.VMEM((cpt, CH), jnp.float32),
            pltpu.VMEM((npad,), jnp.float32),
            pltpu.VMEM((npad,), jnp.float32),
            pltpu.VMEM((CH, D), jnp.float32),
            pltpu.VMEM((CH,), jnp.float32),
            pltpu.VMEM((64, D), jnp.float32),
            pltpu.VMEM_SHARED((npad, D), jnp.float32),
            pltpu.SemaphoreType.DMA,
        ],
    )
    def spmm_kernel(xf_h, src_h, dst_h, ew_h, ns_h, nd_h, p_h,
                    src_v, dst_v, ew_v, ns_v, nd_v, rows_v, w_v, zbuf,
                    acc, sem):
        ZBr = 64
        cid = lax.axis_index("c")
        sid = lax.axis_index("s")
        wid = cid * NS + sid
        pltpu.sync_copy(src_h.at[wid], src_v)
        pltpu.sync_copy(dst_h.at[wid], dst_v)
        pltpu.sync_copy(ew_h.at[wid], ew_v)
        pltpu.sync_copy(ns_h, ns_v)
        pltpu.sync_copy(nd_h, nd_v)

        zv = jnp.zeros((L,), jnp.float32)

        def zrow(i, _):
            for dcol in range(D // L):
                zbuf[i, pl.ds(dcol * L, L)] = zv
            return 0
        lax.fori_loop(0, ZBr, zrow, 0)
        base = sid * rows_pt

        def zacc(k, _):
            pltpu.sync_copy(zbuf, acc.at[pl.ds(base + k * ZBr, ZBr)])
            return 0
        lax.fori_loop(0, nz, zacc, 0)

        plsc.subcore_barrier()

        def chunk(j, _):
            pltpu.async_copy(xf_h.at[src_v.at[j]], rows_v, sem).wait()
            for r in range(CH // L):
                sl = pl.ds(r * L, L)
                s_idx = src_v[j, sl]
                d_idx = dst_v[j, sl]
                w = (plsc.load_gather(ns_v, [s_idx])
                     * plsc.load_gather(nd_v, [d_idx])
                     * ew_v[j, sl])
                w_v[sl] = w

            def scale(e, _):
                we = w_v[e]
                for dcol in range(D // L):
                    sl = pl.ds(dcol * L, L)
                    rows_v[e, sl] = rows_v[e, sl] * we
                return 0
            lax.fori_loop(0, CH, scale, 0)
            pltpu.sync_copy(rows_v, acc.at[dst_v.at[j]], add=True)
            return 0
        lax.fori_loop(0, cpt, chunk, 0)

        plsc.subcore_barrier()

        def dump(k, _):
            pltpu.sync_copy(acc.at[pl.ds(base + k * ZBr, ZBr)], zbuf)
            pltpu.sync_copy(zbuf, p_h.at[cid, pl.ds(base + k * ZBr, ZBr)])
            return 0
        lax.fori_loop(0, nz, dump, 0)

    return spmm_kernel(x_h, src3, dst3, ew3, ns, nd)


def _dense_body(relu_scale, p_ref, nin_ref, nout_ref, w_ref, b_ref, o_ref):
    agg = (p_ref[0] + p_ref[1]) * nin_ref[...][:, None]
    y = lax.dot_general(agg, w_ref[...], (((1,), (1,)), ((), ())),
                        preferred_element_type=jnp.float32)
    y = y + b_ref[...][None, :]
    if relu_scale:
        y = jnp.maximum(y, 0.0) * nout_ref[...][:, None]
    o_ref[...] = y


def _dense(P, nin, nout, W, b, relu_scale):
    npad = P.shape[1]
    BR = 512
    return pl.pallas_call(
        functools.partial(_dense_body, relu_scale),
        grid=(npad // BR,),
        in_specs=[pl.BlockSpec((NC, BR, D), lambda i: (0, i, 0)),
                  pl.BlockSpec((BR,), lambda i: (i,)),
                  pl.BlockSpec((BR,), lambda i: (i,)),
                  pl.BlockSpec((D, D), lambda i: (0, 0)),
                  pl.BlockSpec((D,), lambda i: (0,))],
        out_specs=pl.BlockSpec((BR, D), lambda i: (i, 0)),
        out_shape=jax.ShapeDtypeStruct((npad, D), jnp.float32),
    )(P, nin, nout, W, b)


def kernel(node_feats, edge_index, edge_feats, W_e, b_e, W1, b1, W2, b2):
    n = node_feats.shape[0]
    E = edge_feats.shape[0]
    npad = ((n + 1 + 1023) // 1024) * 1024
    EP = ((E + NW * CH - 1) // (NW * CH)) * NW * CH
    cpt = EP // (NW * CH)

    src = edge_index[0]
    dst = edge_index[1]
    padi = jnp.full((EP - E,), n, jnp.int32)
    src3 = jnp.concatenate([src, padi]).reshape(NW, cpt, CH)
    dst3 = jnp.concatenate([dst, padi]).reshape(NW, cpt, CH)

    ew = _compute_ew(edge_feats, W_e, b_e)
    ew3 = jnp.concatenate([ew, jnp.ones((EP - E,), jnp.float32)]
                          ).reshape(NW, cpt, CH)

    xp = jnp.zeros((npad, D), jnp.float32).at[:n].set(node_feats)

    tabs = _sc_degrees(src3, dst3, ew3, npad)
    ns, nd, nin, nout, x1 = _compute_norms(tabs, xp)

    P1 = _sc_spmm(x1, src3, dst3, ew3, ns, nd, npad)
    x2 = _dense(P1, nin, nout, W1, b1, relu_scale=True)
    P2 = _sc_spmm(x2, src3, dst3, ew3, ns, nd, npad)
    out = _dense(P2, nin, nout, W2, b2, relu_scale=False)
    return out[:n]


# speedup vs baseline: 6.0414x; 6.0414x over previous
import functools

import jax
import jax.numpy as jnp
from jax import lax
from jax.experimental import pallas as pl
from jax.experimental.pallas import tpu as pltpu
from jax.experimental.pallas import tpu_sc as plsc

D = 128
D_EDGE = 16
NC, NS, L = 2, 16, 16
NW = NC * NS
CH = 128


def _ew_body(ef_ref, we_ref, be_ref, out_ref):
    out_ref[...] = jnp.sum(ef_ref[...] * we_ref[...], axis=1) + be_ref[0]


def _compute_ew(edge_feats, W_e, b_e):
    E = edge_feats.shape[0]
    BE = 512
    assert E % BE == 0
    return pl.pallas_call(
        _ew_body,
        grid=(E // BE,),
        in_specs=[pl.BlockSpec((BE, D_EDGE), lambda i: (i, 0)),
                  pl.BlockSpec((1, D_EDGE), lambda i: (0, 0)),
                  pl.BlockSpec((1,), lambda i: (0,))],
        out_specs=pl.BlockSpec((BE,), lambda i: (i,)),
        out_shape=jax.ShapeDtypeStruct((E,), jnp.float32),
    )(edge_feats, W_e, b_e)


def _sc_degrees(src3, dst3, ew3, npad):
    cpt = src3.shape[1]
    mesh = plsc.VectorSubcoreMesh(core_axis_name="c", subcore_axis_name="s")

    @functools.partial(
        pl.kernel,
        out_type=jax.ShapeDtypeStruct((NW, 4, npad), jnp.float32),
        mesh=mesh,
        compiler_params=pltpu.CompilerParams(needs_layout_passes=False),
        scratch_types=[
            pltpu.VMEM((cpt, CH), jnp.int32),
            pltpu.VMEM((cpt, CH), jnp.int32),
            pltpu.VMEM((cpt, CH), jnp.float32),
            pltpu.VMEM((npad,), jnp.float32),
            pltpu.VMEM((npad,), jnp.float32),
            pltpu.VMEM((npad,), jnp.float32),
            pltpu.VMEM((npad,), jnp.float32),
        ],
    )
    def deg_kernel(src_h, dst_h, ew_h, tabs_h,
                   src_v, dst_v, ew_v, tow, tdo, tiw, tdi):
        cid = lax.axis_index("c")
        sid = lax.axis_index("s")
        wid = cid * NS + sid
        pltpu.sync_copy(src_h.at[wid], src_v)
        pltpu.sync_copy(dst_h.at[wid], dst_v)
        pltpu.sync_copy(ew_h.at[wid], ew_v)

        zv = jnp.zeros((L,), jnp.float32)

        def ztab(i, _):
            sl = pl.ds(i * L, L)
            tow[sl] = zv
            tdo[sl] = zv
            tiw[sl] = zv
            tdi[sl] = zv
            return 0
        lax.fori_loop(0, npad // L, ztab, 0)

        ones = jnp.ones((L,), jnp.float32)

        def chunk(j, _):
            def group(r, _):
                sl = pl.ds(r * L, L)
                s = src_v[j, sl]
                d = dst_v[j, sl]
                v = ew_v[j, sl]
                plsc.addupdate_scatter(tow, [s], v)
                plsc.addupdate_scatter(tdo, [s], ones)
                plsc.addupdate_scatter(tiw, [d], v)
                plsc.addupdate_scatter(tdi, [d], ones)
                return 0
            lax.fori_loop(0, CH // L, group, 0)
            return 0
        lax.fori_loop(0, cpt, chunk, 0)

        pltpu.sync_copy(tow, tabs_h.at[wid, 0])
        pltpu.sync_copy(tdo, tabs_h.at[wid, 1])
        pltpu.sync_copy(tiw, tabs_h.at[wid, 2])
        pltpu.sync_copy(tdi, tabs_h.at[wid, 3])

    return deg_kernel(src3, dst3, ew3)


def _norms_body(tabs_ref, x_ref, ns_ref, nd_ref, nin_ref, nout_ref, x1_ref):
    t = jnp.sum(tabs_ref[...], axis=0)
    out_w = t[0]
    deg_o = t[1]
    in_w = t[2]
    deg_i = t[3]
    ns_ref[...] = lax.rsqrt(out_w)
    nd_ref[...] = lax.rsqrt(in_w)
    nin_ref[...] = lax.rsqrt(jnp.maximum(deg_i, 1.0))
    nout = lax.rsqrt(jnp.maximum(deg_o, 1.0))
    nout_ref[...] = nout
    x1_ref[...] = x_ref[...] * nout[:, None]


def _compute_norms(tabs, xp):
    npad = xp.shape[0]
    BN = 1024
    vec = jax.ShapeDtypeStruct((npad,), jnp.float32)
    return pl.pallas_call(
        _norms_body,
        grid=(npad // BN,),
        in_specs=[pl.BlockSpec((NW, 4, BN), lambda i: (0, 0, i)),
                  pl.BlockSpec((BN, D), lambda i: (i, 0))],
        out_specs=[pl.BlockSpec((BN,), lambda i: (i,))] * 4
        + [pl.BlockSpec((BN, D), lambda i: (i, 0))],
        out_shape=[vec, vec, vec, vec,
                   jax.ShapeDtypeStruct((npad, D), jnp.float32)],
    )(tabs, xp)


def _sc_wnorm(src3, dst3, ew3, ns, nd, npad):
    cpt = src3.shape[1]
    mesh = plsc.VectorSubcoreMesh(core_axis_name="c", subcore_axis_name="s")

    @functools.partial(
        pl.kernel,
        out_type=jax.ShapeDtypeStruct((NW, cpt, CH), jnp.float32),
        mesh=mesh,
        compiler_params=pltpu.CompilerParams(needs_layout_passes=False),
        scratch_types=[
            pltpu.VMEM((cpt, CH), jnp.int32),
            pltpu.VMEM((cpt, CH), jnp.int32),
            pltpu.VMEM((cpt, CH), jnp.float32),
            pltpu.VMEM((cpt, CH), jnp.float32),
            pltpu.VMEM((npad,), jnp.float32),
            pltpu.VMEM((npad,), jnp.float32),
        ],
    )
    def wnorm_kernel(src_h, dst_h, ew_h, ns_h, nd_h, w_out,
                     src_v, dst_v, ew_v, w_v, ns_v, nd_v):
        cid = lax.axis_index("c")
        sid = lax.axis_index("s")
        wid = cid * NS + sid
        pltpu.sync_copy(src_h.at[wid], src_v)
        pltpu.sync_copy(dst_h.at[wid], dst_v)
        pltpu.sync_copy(ew_h.at[wid], ew_v)
        pltpu.sync_copy(ns_h, ns_v)
        pltpu.sync_copy(nd_h, nd_v)

        def chunk(j, _):
            def group(r, _):
                sl = pl.ds(r * L, L)
                w = (plsc.load_gather(ns_v, [src_v[j, sl]])
                     * plsc.load_gather(nd_v, [dst_v[j, sl]])
                     * ew_v[j, sl])
                w_v[j, sl] = w
                return 0
            lax.fori_loop(0, CH // L, group, 0)
            return 0
        lax.fori_loop(0, cpt, chunk, 0)
        pltpu.sync_copy(w_v, w_out.at[wid])

    return wnorm_kernel(src3, dst3, ew3, ns, nd)


def _sc_spmm(x_h, src3, dst3, w3, npad):
    cpt = src3.shape[1]
    rows_pt = npad // NS
    nz = rows_pt // CH
    mesh = plsc.VectorSubcoreMesh(core_axis_name="c", subcore_axis_name="s")

    @functools.partial(
        pl.kernel,
        out_type=jax.ShapeDtypeStruct((NC, npad, D), jnp.float32),
        mesh=mesh,
        compiler_params=pltpu.CompilerParams(needs_layout_passes=False),
        scratch_types=[
            pltpu.VMEM((cpt, CH), jnp.int32),
            pltpu.VMEM((cpt, CH), jnp.int32),
            pltpu.VMEM((cpt, CH), jnp.float32),
            pltpu.VMEM((CH, D), jnp.float32),
            pltpu.VMEM_SHARED((npad, D), jnp.float32),
            pltpu.SemaphoreType.DMA,
        ],
    )
    def spmm_kernel(xf_h, src_h, dst_h, w_h, p_h,
                    src_v, dst_v, w_v, rows_v, acc, sem):
        cid = lax.axis_index("c")
        sid = lax.axis_index("s")
        wid = cid * NS + sid
        pltpu.sync_copy(src_h.at[wid], src_v)
        pltpu.sync_copy(dst_h.at[wid], dst_v)
        pltpu.sync_copy(w_h.at[wid], w_v)

        zv = jnp.zeros((L,), jnp.float32)

        def zrow(i, _):
            for dcol in range(D // L):
                rows_v[i, pl.ds(dcol * L, L)] = zv
            return 0
        lax.fori_loop(0, CH, zrow, 0)
        base = sid * rows_pt

        def zacc(k, _):
            pltpu.sync_copy(rows_v, acc.at[pl.ds(base + k * CH, CH)])
            return 0
        lax.fori_loop(0, nz, zacc, 0)

        plsc.subcore_barrier()

        def chunk(j, _):
            pltpu.async_copy(xf_h.at[src_v.at[j]], rows_v, sem).wait()

            def group(r, _):
                w = w_v[j, pl.ds(r * L, L)]
                for e in range(L):
                    we = w[e]
                    row = r * L + e
                    for dcol in range(D // L):
                        fsl = pl.ds(dcol * L, L)
                        rows_v[row, fsl] = rows_v[row, fsl] * we
                return 0
            lax.fori_loop(0, CH // L, group, 0)
            pltpu.sync_copy(rows_v, acc.at[dst_v.at[j]], add=True)
            return 0
        lax.fori_loop(0, cpt, chunk, 0)

        plsc.subcore_barrier()

        def dump(k, _):
            pltpu.sync_copy(acc.at[pl.ds(base + k * CH, CH)], rows_v)
            pltpu.sync_copy(rows_v, p_h.at[cid, pl.ds(base + k * CH, CH)])
            return 0
        lax.fori_loop(0, nz, dump, 0)

    return spmm_kernel(x_h, src3, dst3, w3)


def _dense_body(relu_scale, p_ref, nin_ref, nout_ref, w_ref, b_ref, o_ref):
    agg = (p_ref[0] + p_ref[1]) * nin_ref[...][:, None]
    y = lax.dot_general(agg, w_ref[...], (((1,), (1,)), ((), ())),
                        preferred_element_type=jnp.float32)
    y = y + b_ref[...][None, :]
    if relu_scale:
        y = jnp.maximum(y, 0.0) * nout_ref[...][:, None]
    o_ref[...] = y


def _dense(P, nin, nout, W, b, relu_scale):
    npad = P.shape[1]
    BR = 512
    return pl.pallas_call(
        functools.partial(_dense_body, relu_scale),
        grid=(npad // BR,),
        in_specs=[pl.BlockSpec((NC, BR, D), lambda i: (0, i, 0)),
                  pl.BlockSpec((BR,), lambda i: (i,)),
                  pl.BlockSpec((BR,), lambda i: (i,)),
                  pl.BlockSpec((D, D), lambda i: (0, 0)),
                  pl.BlockSpec((D,), lambda i: (0,))],
        out_specs=pl.BlockSpec((BR, D), lambda i: (i, 0)),
        out_shape=jax.ShapeDtypeStruct((npad, D), jnp.float32),
    )(P, nin, nout, W, b)


def kernel(node_feats, edge_index, edge_feats, W_e, b_e, W1, b1, W2, b2):
    n = node_feats.shape[0]
    E = edge_feats.shape[0]
    npad = ((n + 1 + 1023) // 1024) * 1024
    EP = ((E + NW * CH - 1) // (NW * CH)) * NW * CH
    cpt = EP // (NW * CH)

    src = edge_index[0]
    dst = edge_index[1]
    padi = jnp.full((EP - E,), n, jnp.int32)
    src3 = jnp.concatenate([src, padi]).reshape(NW, cpt, CH)
    dst3 = jnp.concatenate([dst, padi]).reshape(NW, cpt, CH)

    ew = _compute_ew(edge_feats, W_e, b_e)
    ew3 = jnp.concatenate([ew, jnp.ones((EP - E,), jnp.float32)]
                          ).reshape(NW, cpt, CH)

    xp = jnp.zeros((npad, D), jnp.float32).at[:n].set(node_feats)

    tabs = _sc_degrees(src3, dst3, ew3, npad)
    ns, nd, nin, nout, x1 = _compute_norms(tabs, xp)
    w3 = _sc_wnorm(src3, dst3, ew3, ns, nd, npad)

    P1 = _sc_spmm(x1, src3, dst3, w3, npad)
    x2 = _dense(P1, nin, nout, W1, b1, relu_scale=True)
    P2 = _sc_spmm(x2, src3, dst3, w3, npad)
    out = _dense(P2, nin, nout, W2, b2, relu_scale=False)
    return out[:n]
